# Initial kernel scaffold; baseline (speedup 1.0000x reference)
import functools

import jax
import jax.numpy as jnp
from jax import lax
from jax.experimental import pallas as pl
from jax.experimental.pallas import tpu as pltpu

B, S, D = 4, 4096, 4096
_BS = 1024


def _mv_body(x_ref, w_ref, o_ref):
    o_ref[...] = jnp.dot(
        x_ref[...], w_ref[...], preferred_element_type=jnp.float32
    )


def _matvec(x2d, w2d):
    n = x2d.shape[0]
    return pl.pallas_call(
        _mv_body,
        grid=(n // _BS,),
        in_specs=[
            pl.BlockSpec((_BS, D), lambda i: (i, 0)),
            pl.BlockSpec((D, 1), lambda i: (0, 0)),
        ],
        out_specs=pl.BlockSpec((_BS, 1), lambda i: (i, 0)),
        out_shape=jax.ShapeDtypeStruct((n, 1), jnp.float32),
    )(x2d, w2d)


def _fin_body(lg_ref, sot_ref, lab_ref, b_ref, loss_ref, pred_ref):
    lg = lg_ref[...] + b_ref[...]
    mask = sot_ref[...] != 0
    neg_inf = jnp.float32(-jnp.inf)
    ml = jnp.where(mask, lg, neg_inf)
    m = jnp.max(ml, axis=1, keepdims=True)
    m_safe = jnp.where(m == neg_inf, jnp.float32(0.0), m)
    su = jnp.sum(jnp.exp(ml - m_safe), axis=1, keepdims=True)
    lse = m_safe + jnp.log(su)

    iota = lax.broadcasted_iota(jnp.int32, (B, S), 1)
    big = jnp.int32(1 << 30)
    hit = ml == m
    idx = jnp.min(jnp.where(hit, iota, big), axis=1, keepdims=True)

    mi = mask.astype(jnp.int32)
    rank = jnp.cumsum(mi, axis=1) - 1
    pred = jnp.sum(jnp.where(iota == idx, rank, 0), axis=1, keepdims=True)

    lab = lab_ref[...]
    sel = mask & (rank == lab)
    exists = jnp.sum(sel.astype(jnp.int32), axis=1, keepdims=True) > 0
    chosen_sel = jnp.sum(jnp.where(sel, lg, jnp.float32(0.0)), axis=1,
                         keepdims=True)
    chosen = jnp.where(exists, chosen_sel, lg[:, 0:1])
    loss = jnp.sum(lse - chosen) * jnp.float32(1.0 / B)
    loss_ref[...] = loss.reshape(1, 1)
    pred_ref[...] = pred


def _finish(lg, sot, labels, b):
    return pl.pallas_call(
        _fin_body,
        in_specs=[
            pl.BlockSpec((B, S), lambda: (0, 0)),
            pl.BlockSpec((B, S), lambda: (0, 0)),
            pl.BlockSpec((B, 1), lambda: (0, 0)),
            pl.BlockSpec((1, 1), lambda: (0, 0)),
        ],
        out_specs=(
            pl.BlockSpec((1, 1), lambda: (0, 0)),
            pl.BlockSpec((B, 1), lambda: (0, 0)),
        ),
        out_shape=(
            jax.ShapeDtypeStruct((1, 1), jnp.float32),
            jax.ShapeDtypeStruct((B, 1), jnp.int32),
        ),
    )(lg, sot, labels.reshape(B, 1), b.reshape(1, 1))


def kernel(sequence_output, sot_positions, labels, w, b):
    x2d = sequence_output.reshape(B * S, D)
    lg = _matvec(x2d, w.reshape(D, 1)).reshape(B, S)
    loss, pred = _finish(lg, sot_positions, labels, b)
    return loss.reshape(()), pred.reshape(B), labels


# speedup vs baseline: 1.4316x; 1.4316x over previous
import functools

import jax
import jax.numpy as jnp
from jax import lax
from jax.experimental import pallas as pl
from jax.experimental.pallas import tpu as pltpu

B, S, D = 4, 4096, 4096
_BS = 1024


def _mv_body(x_ref, w_ref, o_ref):
    o_ref[...] = jnp.dot(
        x_ref[...], w_ref[...], preferred_element_type=jnp.float32
    )


def _matvec(x2d, w2d):
    n = x2d.shape[0]
    return pl.pallas_call(
        _mv_body,
        grid=(n // _BS,),
        in_specs=[
            pl.BlockSpec((_BS, D), lambda i: (i, 0)),
            pl.BlockSpec((D, 1), lambda i: (0, 0)),
        ],
        out_specs=pl.BlockSpec((_BS, 1), lambda i: (i, 0)),
        out_shape=jax.ShapeDtypeStruct((n, 1), jnp.float32),
    )(x2d, w2d)


def _fin_body(lg_ref, sot_ref, lab_ref, b_ref, loss_ref, pred_ref):
    lg = lg_ref[...] + b_ref[...]
    mask = sot_ref[...] != 0
    neg_inf = jnp.float32(-jnp.inf)
    ml = jnp.where(mask, lg, neg_inf)
    m = jnp.max(ml, axis=1, keepdims=True)
    m_safe = jnp.where(m == neg_inf, jnp.float32(0.0), m)
    su = jnp.sum(jnp.exp(ml - m_safe), axis=1, keepdims=True)
    lse = m_safe + jnp.log(su)

    iota = lax.broadcasted_iota(jnp.int32, (B, S), 1)
    big = jnp.int32(1 << 30)
    hit = ml == m
    idx = jnp.min(jnp.where(hit, iota, big), axis=1, keepdims=True)

    mi = mask.astype(jnp.int32)
    r = mi
    sh = 1
    while sh < S:
        zeros = jnp.zeros((B, sh), jnp.int32)
        r = r + jnp.concatenate([zeros, r[:, : S - sh]], axis=1)
        sh *= 2
    rank = r - 1
    pred = jnp.sum(jnp.where(iota == idx, rank, 0), axis=1, keepdims=True)

    lab = lab_ref[...]
    sel = mask & (rank == lab)
    exists = jnp.sum(sel.astype(jnp.int32), axis=1, keepdims=True) > 0
    chosen_sel = jnp.sum(jnp.where(sel, lg, jnp.float32(0.0)), axis=1,
                         keepdims=True)
    chosen = jnp.where(exists, chosen_sel, lg[:, 0:1])
    loss = jnp.sum(lse - chosen) * jnp.float32(1.0 / B)
    loss_ref[...] = loss.reshape(1, 1)
    pred_ref[...] = pred


def _finish(lg, sot, labels, b):
    return pl.pallas_call(
        _fin_body,
        in_specs=[
            pl.BlockSpec((B, S), lambda: (0, 0)),
            pl.BlockSpec((B, S), lambda: (0, 0)),
            pl.BlockSpec((B, 1), lambda: (0, 0)),
            pl.BlockSpec((1, 1), lambda: (0, 0)),
        ],
        out_specs=(
            pl.BlockSpec((1, 1), lambda: (0, 0)),
            pl.BlockSpec((B, 1), lambda: (0, 0)),
        ),
        out_shape=(
            jax.ShapeDtypeStruct((1, 1), jnp.float32),
            jax.ShapeDtypeStruct((B, 1), jnp.int32),
        ),
    )(lg, sot, labels.reshape(B, 1), b.reshape(1, 1))


def kernel(sequence_output, sot_positions, labels, w, b):
    x2d = sequence_output.reshape(B * S, D)
    lg = _matvec(x2d, w.reshape(D, 1)).reshape(B, S)
    loss, pred = _finish(lg, sot_positions, labels, b)
    return loss.reshape(()), pred.reshape(B), labels
